# baseline (speedup 1.0000x reference)
import jax
import jax.numpy as jnp
from jax import lax
from jax.experimental import pallas as pl
from jax.experimental.pallas import tpu as pltpu
from jax.experimental.pallas import tpu_sc as plsc

IMG_W = 512
IMG_H = 512
BATCH = 64
NOUT = 129540
NOUT_PAD = NOUT + 124
NWORKERS = 32
PAIR_OUT = 509
REG_PAIRS = 8
REG_IN = 16 * IMG_W
REG_OUT = REG_PAIRS * PAIR_OUT
TAIL_PAIRS = 7
TAIL_IN = 15 * IMG_W
TAIL_OUT = 6 * PAIR_OUT + 254 + 4
IN_BUF = REG_IN + 16
OUT_BUF = REG_OUT + 16


def _sc_body(x_ref, out_ref, inb0, inb1, outb0, outb1, is0, is1, os0, os1):
  nc = 2
  wid = lax.axis_index("s") * nc + lax.axis_index("c")

  iota = lax.iota(jnp.int32, 16)
  two_iota = iota * 2
  mixed15 = two_iota + 480 + jnp.where(
      iota < 14, jnp.full((16,), 2, jnp.int32), jnp.full((16,), 5, jnp.int32))

  inbs = (inb0, inb1)
  outbs = (outb0, outb1)
  isems = (is0, is1)
  osems = (os0, os1)

  def compute(inb, outb, npairs):
    for p in range(npairs):
      pb = 1024 * p
      for j in range(32):
        if j == 15:
          idx = mixed15 + pb
        else:
          c = 2 if j < 15 else 5
          idx = two_iota + (pb + 32 * j + c)
        v = plsc.load_gather(inb, [idx])
        outb[pl.ds(PAIR_OUT * p + 16 * j, 16)] = v

  def run(npairs, in_len, out_len):
    in_off = IMG_W * (16 * wid + 1)
    out_off = REG_OUT * wid

    def fire_in(b, d):
      pltpu.make_async_copy(x_ref.at[b, pl.ds(in_off, in_len)],
                            inbs[d].at[pl.ds(0, in_len)], isems[d]).start()

    def wait_in(d):
      pltpu.make_async_copy(x_ref.at[0, pl.ds(0, in_len)],
                            inbs[d].at[pl.ds(0, in_len)], isems[d]).wait()

    def fire_out(b, d):
      pltpu.make_async_copy(outbs[d].at[pl.ds(0, out_len)],
                            out_ref.at[b, pl.ds(out_off, out_len)],
                            osems[d]).start()

    def wait_out(d):
      pltpu.make_async_copy(x_ref.at[0, pl.ds(0, out_len)],
                            outbs[d].at[pl.ds(0, out_len)], osems[d]).wait()

    fire_in(0, 0)
    fire_in(1, 1)

    def step(i, carry):
      for d in range(2):
        b = 2 * i + d
        wait_in(d)
        pl.when(i >= 1)(lambda: wait_out(d))
        compute(inbs[d], outbs[d], npairs)
        fire_out(b, d)
        pl.when(i <= (BATCH // 2 - 2))(lambda: fire_in(b + 2, d))
      return carry

    lax.fori_loop(0, BATCH // 2, step, 0)
    wait_out(0)
    wait_out(1)

  pl.when(wid < NWORKERS - 1)(lambda: run(REG_PAIRS, REG_IN, REG_OUT))
  pl.when(wid == NWORKERS - 1)(lambda: run(TAIL_PAIRS, TAIL_IN, TAIL_OUT))


_COPY_BLK = 8064
_COPY_GRID_C = 17
_LAST_LN = 640


def _tc_copy_body(src_ref, dst_ref, sem):
  g = pl.program_id(0)
  c = pl.program_id(1)

  def do(ln):
    def _():
      cps = []
      for i in range(8):
        off = (8 * g + i) * NOUT_PAD + _COPY_BLK * c
        cp = pltpu.make_async_copy(src_ref.at[pl.ds(off, ln)],
                                   dst_ref.at[i, pl.ds(0, ln)], sem)
        cp.start()
        cps.append(cp)
      for cp in cps:
        cp.wait()
    return _

  pl.when(c < _COPY_GRID_C - 1)(do(_COPY_BLK))
  pl.when(c == _COPY_GRID_C - 1)(do(_LAST_LN))


@jax.jit
def _line_filter(xf):
  mesh = plsc.VectorSubcoreMesh(core_axis_name="c", subcore_axis_name="s")
  return pl.kernel(
      _sc_body,
      out_type=jax.ShapeDtypeStruct((BATCH, NOUT), jnp.float32),
      mesh=mesh,
      compiler_params=pltpu.CompilerParams(
          use_tc_tiling_on_sc=False, needs_layout_passes=False,
          disable_bounds_checks=True),
      scratch_types=[
          pltpu.VMEM((IN_BUF,), jnp.float32),
          pltpu.VMEM((IN_BUF,), jnp.float32),
          pltpu.VMEM((OUT_BUF,), jnp.float32),
          pltpu.VMEM((OUT_BUF,), jnp.float32),
          pltpu.SemaphoreType.DMA,
          pltpu.SemaphoreType.DMA,
          pltpu.SemaphoreType.DMA,
          pltpu.SemaphoreType.DMA,
      ],
  )(xf)


def kernel(x):
  xf = x.reshape(BATCH, IMG_H * IMG_W)
  return _line_filter(xf)


# speedup vs baseline: 2.1489x; 2.1489x over previous
import jax
import jax.numpy as jnp
from jax import lax
from jax.experimental import pallas as pl
from jax.experimental.pallas import tpu as pltpu
from jax.experimental.pallas import tpu_sc as plsc

IMG_W = 512
IMG_H = 512
BATCH = 64
NOUT = 129540
NOUT_PAD = 130048
NWORKERS = 32
PAIR_OUT = 509
REG_PAIRS = 8
REG_IN = 16 * IMG_W
REG_OUT = REG_PAIRS * PAIR_OUT
TAIL_PAIRS = 7
TAIL_IN = 15 * IMG_W
TAIL_OUT = 6 * PAIR_OUT + 254 + 4
IN_BUF = REG_IN + 16
OUT_BUF = REG_OUT + 16


def _sc_body(x_ref, out_ref, inb0, inb1, outb0, outb1, is0, is1, os0, os1):
  nc = 2
  wid = lax.axis_index("s") * nc + lax.axis_index("c")

  iota = lax.iota(jnp.int32, 16)
  two_iota = iota * 2
  mixed15 = two_iota + 480 + jnp.where(
      iota < 14, jnp.full((16,), 2, jnp.int32), jnp.full((16,), 5, jnp.int32))

  inbs = (inb0, inb1)
  outbs = (outb0, outb1)
  isems = (is0, is1)
  osems = (os0, os1)

  def compute(inb, outb, npairs):
    for p in range(npairs):
      pb = 1024 * p
      for j in range(32):
        if j == 15:
          idx = mixed15 + pb
        else:
          c = 2 if j < 15 else 5
          idx = two_iota + (pb + 32 * j + c)
        v = plsc.load_gather(inb, [idx])
        outb[pl.ds(PAIR_OUT * p + 16 * j, 16)] = v

  def run(npairs, in_len, out_len):
    in_off = IMG_W * (16 * wid + 1)
    out_off = REG_OUT * wid

    def fire_in(b, d):
      pltpu.make_async_copy(x_ref.at[b, pl.ds(in_off, in_len)],
                            inbs[d].at[pl.ds(0, in_len)], isems[d]).start()

    def wait_in(d):
      pltpu.make_async_copy(x_ref.at[0, pl.ds(0, in_len)],
                            inbs[d].at[pl.ds(0, in_len)], isems[d]).wait()

    def fire_out(b, d):
      pltpu.make_async_copy(outbs[d].at[pl.ds(0, out_len)],
                            out_ref.at[pl.ds(b * NOUT_PAD + out_off, out_len)],
                            osems[d]).start()

    def wait_out(d):
      pltpu.make_async_copy(x_ref.at[0, pl.ds(0, out_len)],
                            outbs[d].at[pl.ds(0, out_len)], osems[d]).wait()

    fire_in(0, 0)
    fire_in(1, 1)

    def step(i, carry):
      for d in range(2):
        b = 2 * i + d
        wait_in(d)
        pl.when(i >= 1)(lambda: wait_out(d))
        compute(inbs[d], outbs[d], npairs)
        fire_out(b, d)
        pl.when(i <= (BATCH // 2 - 2))(lambda: fire_in(b + 2, d))
      return carry

    lax.fori_loop(0, BATCH // 2, step, 0)
    wait_out(0)
    wait_out(1)

  pl.when(wid < NWORKERS - 1)(lambda: run(REG_PAIRS, REG_IN, REG_OUT))
  pl.when(wid == NWORKERS - 1)(lambda: run(TAIL_PAIRS, TAIL_IN, TAIL_OUT))


_COPY_SL = 128
_COPY_BLK = _COPY_SL * 128


def _tc_copy_body(src_ref, dst_ref):
  dst_ref[...] = src_ref[...].reshape(8, _COPY_BLK)


@jax.jit
def _line_filter(xf):
  mesh = plsc.VectorSubcoreMesh(core_axis_name="c", subcore_axis_name="s")
  staged = pl.kernel(
      _sc_body,
      out_type=jax.ShapeDtypeStruct((BATCH * NOUT_PAD,), jnp.float32),
      mesh=mesh,
      compiler_params=pltpu.CompilerParams(
          use_tc_tiling_on_sc=False, needs_layout_passes=False),
      scratch_types=[
          pltpu.VMEM((IN_BUF,), jnp.float32),
          pltpu.VMEM((IN_BUF,), jnp.float32),
          pltpu.VMEM((OUT_BUF,), jnp.float32),
          pltpu.VMEM((OUT_BUF,), jnp.float32),
          pltpu.SemaphoreType.DMA,
          pltpu.SemaphoreType.DMA,
          pltpu.SemaphoreType.DMA,
          pltpu.SemaphoreType.DMA,
      ],
  )(xf)
  staged3 = staged.reshape(BATCH, NOUT_PAD // 128, 128)
  return pl.pallas_call(
      _tc_copy_body,
      grid=(BATCH // 8, (NOUT + _COPY_BLK - 1) // _COPY_BLK),
      in_specs=[pl.BlockSpec((8, _COPY_SL, 128), lambda g, c: (g, c, 0))],
      out_specs=pl.BlockSpec((8, _COPY_BLK), lambda g, c: (g, c)),
      out_shape=jax.ShapeDtypeStruct((BATCH, NOUT), jnp.float32),
  )(staged3)


def kernel(x):
  xf = x.reshape(BATCH, IMG_H * IMG_W)
  return _line_filter(xf)


# speedup vs baseline: 2.4324x; 1.1319x over previous
import jax
import jax.numpy as jnp
from jax import lax
from jax.experimental import pallas as pl
from jax.experimental.pallas import tpu as pltpu
from jax.experimental.pallas import tpu_sc as plsc

IMG_W = 512
IMG_H = 512
BATCH = 64
NOUT = 129540
NOUT_PAD = 130048
NWORKERS = 32
PAIR_OUT = 509
REG_PAIRS = 8
REG_IN = 16 * IMG_W
REG_OUT = REG_PAIRS * PAIR_OUT
TAIL_PAIRS = 7
TAIL_IN = 15 * IMG_W
TAIL_OUT = 6 * PAIR_OUT + 254 + 4
IN_BUF = REG_IN + 16
OUT_BUF = REG_OUT + 16


NBUF = 4
PAIR_WIN = 1040


def _sc_body(x_ref, out_ref, *bufs):
  inbs = bufs[0:NBUF]
  outbs = bufs[NBUF:2 * NBUF]
  isems = bufs[2 * NBUF:3 * NBUF]
  osems = bufs[3 * NBUF:4 * NBUF]
  nc = 2
  wid = lax.axis_index("s") * nc + lax.axis_index("c")

  iota = lax.iota(jnp.int32, 16)
  two_iota = iota * 2
  mixed15 = two_iota + 480 + jnp.where(
      iota < 14, jnp.full((16,), 2, jnp.int32), jnp.full((16,), 5, jnp.int32))
  idxs = [mixed15 if j == 15 else two_iota + (32 * j + (2 if j < 15 else 5))
          for j in range(32)]

  def compute(inb, outb, npairs):
    for p in range(npairs):
      win = inb.at[pl.ds(1024 * p, PAIR_WIN)]
      for j in range(32):
        v = plsc.load_gather(win, [idxs[j]])
        outb[pl.ds(PAIR_OUT * p + 16 * j, 16)] = v

  def run(npairs, in_len, out_len):
    in_off = IMG_W * (16 * wid + 1)
    out_off = REG_OUT * wid

    def fire_in(b, d):
      pltpu.make_async_copy(x_ref.at[b, pl.ds(in_off, in_len)],
                            inbs[d].at[pl.ds(0, in_len)], isems[d]).start()

    def wait_in(d):
      pltpu.make_async_copy(x_ref.at[0, pl.ds(0, in_len)],
                            inbs[d].at[pl.ds(0, in_len)], isems[d]).wait()

    def fire_out(b, d):
      pltpu.make_async_copy(outbs[d].at[pl.ds(0, out_len)],
                            out_ref.at[pl.ds(b * NOUT_PAD + out_off, out_len)],
                            osems[d]).start()

    def wait_out(d):
      pltpu.make_async_copy(x_ref.at[0, pl.ds(0, out_len)],
                            outbs[d].at[pl.ds(0, out_len)], osems[d]).wait()

    for d in range(NBUF):
      fire_in(d, d)

    def step(i, carry):
      for d in range(NBUF):
        b = NBUF * i + d
        wait_in(d)
        pl.when(i >= 1)(lambda: wait_out(d))
        compute(inbs[d], outbs[d], npairs)
        fire_out(b, d)
        pl.when(i <= (BATCH // NBUF - 2))(lambda: fire_in(b + NBUF, d))
      return carry

    lax.fori_loop(0, BATCH // NBUF, step, 0)
    for d in range(NBUF):
      wait_out(d)

  pl.when(wid < NWORKERS - 1)(lambda: run(REG_PAIRS, REG_IN, REG_OUT))
  pl.when(wid == NWORKERS - 1)(lambda: run(TAIL_PAIRS, TAIL_IN, TAIL_OUT))


_COPY_SL = 128
_COPY_BLK = _COPY_SL * 128


def _tc_copy_body(src_ref, dst_ref):
  dst_ref[...] = src_ref[...].reshape(8, _COPY_BLK)


@jax.jit
def _line_filter(xf):
  mesh = plsc.VectorSubcoreMesh(core_axis_name="c", subcore_axis_name="s")
  staged = pl.kernel(
      _sc_body,
      out_type=jax.ShapeDtypeStruct((BATCH * NOUT_PAD,), jnp.float32),
      mesh=mesh,
      compiler_params=pltpu.CompilerParams(
          use_tc_tiling_on_sc=False, needs_layout_passes=False),
      scratch_types=(
          [pltpu.VMEM((IN_BUF,), jnp.float32)] * NBUF
          + [pltpu.VMEM((OUT_BUF,), jnp.float32)] * NBUF
          + [pltpu.SemaphoreType.DMA] * (2 * NBUF)
      ),
  )(xf)
  staged3 = staged.reshape(BATCH, NOUT_PAD // 128, 128)
  return pl.pallas_call(
      _tc_copy_body,
      grid=(BATCH // 8, (NOUT + _COPY_BLK - 1) // _COPY_BLK),
      in_specs=[pl.BlockSpec((8, _COPY_SL, 128), lambda g, c: (g, c, 0))],
      out_specs=pl.BlockSpec((8, _COPY_BLK), lambda g, c: (g, c)),
      out_shape=jax.ShapeDtypeStruct((BATCH, NOUT), jnp.float32),
  )(staged3)


def kernel(x):
  xf = x.reshape(BATCH, IMG_H * IMG_W)
  return _line_filter(xf)


# speedup vs baseline: 2.6690x; 1.0973x over previous
import jax
import jax.numpy as jnp
from jax import lax
from jax.experimental import pallas as pl
from jax.experimental.pallas import tpu as pltpu
from jax.experimental.pallas import tpu_sc as plsc

IMG_W = 512
IMG_H = 512
BATCH = 64
NOUT = 129540
NOUT_PAD = 130048
NWORKERS = 32
PAIR_OUT = 509
REG_PAIRS = 8
REG_IN = 16 * IMG_W
REG_OUT = REG_PAIRS * PAIR_OUT
TAIL_PAIRS = 7
TAIL_IN = 15 * IMG_W
TAIL_OUT = 6 * PAIR_OUT + 254 + 4
IN_BUF = REG_IN + 16
OUT_BUF = REG_OUT + 16


NBUF = 4
PAIR_WIN = 1040


def _sc_body(x_ref, out_ref, *bufs):
  inbs = bufs[0:NBUF]
  outbs = bufs[NBUF:2 * NBUF]
  isems = bufs[2 * NBUF:3 * NBUF]
  osems = bufs[3 * NBUF:4 * NBUF]
  nc = 2
  wid = lax.axis_index("s") * nc + lax.axis_index("c")

  iota = lax.iota(jnp.int32, 16)
  two_iota = iota * 2
  mixed15 = two_iota + 480 + jnp.where(
      iota < 14, jnp.full((16,), 2, jnp.int32), jnp.full((16,), 5, jnp.int32))
  idxs = [mixed15 if j == 15 else two_iota + (32 * j + (2 if j < 15 else 5))
          for j in range(32)]

  def compute(inb, outb, npairs):
    for p in range(npairs):
      win = inb.at[pl.ds(1024 * p, PAIR_WIN)]
      for j in range(32):
        v = plsc.load_gather(win, [idxs[j]])
        outb[pl.ds(PAIR_OUT * p + 16 * j, 16)] = v

  def run(npairs, in_len, out_len):
    in_off = IMG_W * (16 * wid + 1)
    out_off = REG_OUT * wid

    def fire_in(b, d):
      pltpu.make_async_copy(x_ref.at[b, pl.ds(in_off, in_len)],
                            inbs[d].at[pl.ds(0, in_len)], isems[d]).start()

    def wait_in(d):
      pltpu.make_async_copy(x_ref.at[0, pl.ds(0, in_len)],
                            inbs[d].at[pl.ds(0, in_len)], isems[d]).wait()

    def fire_out(b, d):
      pltpu.make_async_copy(outbs[d].at[pl.ds(0, out_len)],
                            out_ref.at[pl.ds(b * NOUT_PAD + out_off, out_len)],
                            osems[d]).start()

    def wait_out(d):
      pltpu.make_async_copy(x_ref.at[0, pl.ds(0, out_len)],
                            outbs[d].at[pl.ds(0, out_len)], osems[d]).wait()

    for d in range(NBUF):
      fire_in(d, d)

    def step(i, carry):
      for d in range(NBUF):
        b = NBUF * i + d
        wait_in(d)
        pl.when(i >= 1)(lambda: wait_out(d))
        compute(inbs[d], outbs[d], npairs)
        fire_out(b, d)
        pl.when(i <= (BATCH // NBUF - 2))(lambda: fire_in(b + NBUF, d))
      return carry

    lax.fori_loop(0, BATCH // NBUF, step, 0)
    for d in range(NBUF):
      wait_out(d)

  pl.when(wid < NWORKERS - 1)(lambda: run(REG_PAIRS, REG_IN, REG_OUT))
  pl.when(wid == NWORKERS - 1)(lambda: run(TAIL_PAIRS, TAIL_IN, TAIL_OUT))


_COPY_SL = 256
_COPY_BLK = _COPY_SL * 128


def _tc_copy_body(src_ref, dst_ref):
  dst_ref[...] = src_ref[...].reshape(8, _COPY_BLK)


@jax.jit
def _line_filter(xf):
  mesh = plsc.VectorSubcoreMesh(core_axis_name="c", subcore_axis_name="s")
  staged = pl.kernel(
      _sc_body,
      out_type=jax.ShapeDtypeStruct((BATCH * NOUT_PAD,), jnp.float32),
      mesh=mesh,
      compiler_params=pltpu.CompilerParams(
          use_tc_tiling_on_sc=False, needs_layout_passes=False),
      scratch_types=(
          [pltpu.VMEM((IN_BUF,), jnp.float32)] * NBUF
          + [pltpu.VMEM((OUT_BUF,), jnp.float32)] * NBUF
          + [pltpu.SemaphoreType.DMA] * (2 * NBUF)
      ),
  )(xf)
  staged3 = staged.reshape(BATCH, NOUT_PAD // 128, 128)
  return pl.pallas_call(
      _tc_copy_body,
      grid=(BATCH // 8, (NOUT + _COPY_BLK - 1) // _COPY_BLK),
      in_specs=[pl.BlockSpec((8, _COPY_SL, 128), lambda g, c: (g, c, 0))],
      out_specs=pl.BlockSpec((8, _COPY_BLK), lambda g, c: (g, c)),
      out_shape=jax.ShapeDtypeStruct((BATCH, NOUT), jnp.float32),
  )(staged3)


def kernel(x):
  xf = x.reshape(BATCH, IMG_H * IMG_W)
  return _line_filter(xf)


# speedup vs baseline: 2.9130x; 1.0914x over previous
import jax
import jax.numpy as jnp
from jax import lax
from jax.experimental import pallas as pl
from jax.experimental.pallas import tpu as pltpu
from jax.experimental.pallas import tpu_sc as plsc

IMG_W = 512
IMG_H = 512
BATCH = 64
NOUT = 129540
NOUT_PAD = 130048
NWORKERS = 32
PAIR_OUT = 509
REG_PAIRS = 8
REG_IN = 16 * IMG_W
REG_OUT = REG_PAIRS * PAIR_OUT
TAIL_PAIRS = 7
TAIL_IN = 15 * IMG_W
TAIL_OUT = 6 * PAIR_OUT + 254 + 4
IN_BUF = REG_IN + 16
OUT_BUF = REG_OUT + 16


NBUF = 4
PAIR_WIN = 1040


def _sc_body(x_ref, out_ref, *bufs):
  inbs = bufs[0:NBUF]
  outbs = bufs[NBUF:2 * NBUF]
  isems = bufs[2 * NBUF:3 * NBUF]
  osems = bufs[3 * NBUF:4 * NBUF]
  nc = 2
  wid = lax.axis_index("s") * nc + lax.axis_index("c")

  iota = lax.iota(jnp.int32, 16)
  two_iota = iota * 2
  mixed15 = two_iota + 480 + jnp.where(
      iota < 14, jnp.full((16,), 2, jnp.int32), jnp.full((16,), 5, jnp.int32))
  idxs = [mixed15 if j == 15 else two_iota + (32 * j + (2 if j < 15 else 5))
          for j in range(32)]

  def compute(inb, outb, npairs):
    for p in range(npairs):
      win = inb.at[pl.ds(1024 * p, PAIR_WIN)]
      for j in range(32):
        v = plsc.load_gather(win, [idxs[j]])
        outb[pl.ds(PAIR_OUT * p + 16 * j, 16)] = v

  def run(npairs, in_len, out_len):
    in_off = IMG_W * (16 * wid + 1)
    out_off = REG_OUT * wid

    def fire_in(b, d):
      pltpu.make_async_copy(x_ref.at[b, pl.ds(in_off, in_len)],
                            inbs[d].at[pl.ds(0, in_len)], isems[d]).start()

    def wait_in(d):
      pltpu.make_async_copy(x_ref.at[0, pl.ds(0, in_len)],
                            inbs[d].at[pl.ds(0, in_len)], isems[d]).wait()

    def fire_out(b, d):
      pltpu.make_async_copy(outbs[d].at[pl.ds(0, out_len)],
                            out_ref.at[pl.ds(b * NOUT_PAD + out_off, out_len)],
                            osems[d]).start()

    def wait_out(d):
      pltpu.make_async_copy(x_ref.at[0, pl.ds(0, out_len)],
                            outbs[d].at[pl.ds(0, out_len)], osems[d]).wait()

    for d in range(NBUF):
      fire_in(d, d)

    def step(i, carry):
      for d in range(NBUF):
        b = NBUF * i + d
        wait_in(d)
        pl.when(i >= 1)(lambda: wait_out(d))
        compute(inbs[d], outbs[d], npairs)
        fire_out(b, d)
        pl.when(i <= (BATCH // NBUF - 2))(lambda: fire_in(b + NBUF, d))
      return carry

    lax.fori_loop(0, BATCH // NBUF, step, 0)
    for d in range(NBUF):
      wait_out(d)

  pl.when(wid < NWORKERS - 1)(lambda: run(REG_PAIRS, REG_IN, REG_OUT))
  pl.when(wid == NWORKERS - 1)(lambda: run(TAIL_PAIRS, TAIL_IN, TAIL_OUT))


_COPY_SL = 512
_COPY_BLK = _COPY_SL * 128


def _tc_copy_body(src_ref, dst_ref):
  dst_ref[...] = src_ref[...].reshape(8, _COPY_BLK)


@jax.jit
def _line_filter(xf):
  mesh = plsc.VectorSubcoreMesh(core_axis_name="c", subcore_axis_name="s")
  staged = pl.kernel(
      _sc_body,
      out_type=jax.ShapeDtypeStruct((BATCH * NOUT_PAD,), jnp.float32),
      mesh=mesh,
      compiler_params=pltpu.CompilerParams(
          use_tc_tiling_on_sc=False, needs_layout_passes=False),
      scratch_types=(
          [pltpu.VMEM((IN_BUF,), jnp.float32)] * NBUF
          + [pltpu.VMEM((OUT_BUF,), jnp.float32)] * NBUF
          + [pltpu.SemaphoreType.DMA] * (2 * NBUF)
      ),
  )(xf)
  staged3 = staged.reshape(BATCH, NOUT_PAD // 128, 128)
  return pl.pallas_call(
      _tc_copy_body,
      grid=(BATCH // 8, (NOUT + _COPY_BLK - 1) // _COPY_BLK),
      in_specs=[pl.BlockSpec((8, _COPY_SL, 128), lambda g, c: (g, c, 0))],
      out_specs=pl.BlockSpec((8, _COPY_BLK), lambda g, c: (g, c)),
      out_shape=jax.ShapeDtypeStruct((BATCH, NOUT), jnp.float32),
  )(staged3)


def kernel(x):
  xf = x.reshape(BATCH, IMG_H * IMG_W)
  return _line_filter(xf)


# speedup vs baseline: 2.9435x; 1.0105x over previous
import jax
import jax.numpy as jnp
from jax import lax
from jax.experimental import pallas as pl
from jax.experimental.pallas import tpu as pltpu
from jax.experimental.pallas import tpu_sc as plsc

IMG_W = 512
IMG_H = 512
BATCH = 64
NOUT = 129540
NOUT_PAD = 130048
NWORKERS = 32
PAIR_OUT = 509
REG_PAIRS = 8
REG_IN = 16 * IMG_W
REG_OUT = REG_PAIRS * PAIR_OUT
TAIL_PAIRS = 7
TAIL_IN = 15 * IMG_W
TAIL_OUT = 6 * PAIR_OUT + 254 + 4
IN_BUF = REG_IN + 16
OUT_BUF = REG_OUT + 16


NBUF = 4
PAIR_WIN = 1040


def _sc_body(x_ref, out_ref, *bufs):
  inbs = bufs[0:NBUF]
  outbs = bufs[NBUF:2 * NBUF]
  isems = bufs[2 * NBUF:3 * NBUF]
  osems = bufs[3 * NBUF:4 * NBUF]
  nc = 2
  wid = lax.axis_index("s") * nc + lax.axis_index("c")

  iota = lax.iota(jnp.int32, 16)
  two_iota = iota * 2
  mixed15 = two_iota + 480 + jnp.where(
      iota < 14, jnp.full((16,), 2, jnp.int32), jnp.full((16,), 5, jnp.int32))
  idxs = [mixed15 if j == 15 else two_iota + (32 * j + (2 if j < 15 else 5))
          for j in range(32)]

  def compute(inb, outb, npairs):
    for p in range(npairs):
      win = inb.at[pl.ds(1024 * p, PAIR_WIN)]
      for j in range(32):
        v = plsc.load_gather(win, [idxs[j]])
        outb[pl.ds(PAIR_OUT * p + 16 * j, 16)] = v

  def run(npairs, in_len, out_len):
    in_off = IMG_W * (16 * wid + 1)
    out_off = REG_OUT * wid

    def fire_in(b, d):
      pltpu.make_async_copy(x_ref.at[b, pl.ds(in_off, in_len)],
                            inbs[d].at[pl.ds(0, in_len)], isems[d]).start()

    def wait_in(d):
      pltpu.make_async_copy(x_ref.at[0, pl.ds(0, in_len)],
                            inbs[d].at[pl.ds(0, in_len)], isems[d]).wait()

    def fire_out(b, d):
      pltpu.make_async_copy(outbs[d].at[pl.ds(0, out_len)],
                            out_ref.at[pl.ds(b * NOUT_PAD + out_off, out_len)],
                            osems[d]).start()

    def wait_out(d):
      pltpu.make_async_copy(x_ref.at[0, pl.ds(0, out_len)],
                            outbs[d].at[pl.ds(0, out_len)], osems[d]).wait()

    for d in range(NBUF):
      fire_in(d, d)

    def step(i, carry):
      for d in range(NBUF):
        b = NBUF * i + d
        wait_in(d)
        pl.when(i >= 1)(lambda: wait_out(d))
        compute(inbs[d], outbs[d], npairs)
        fire_out(b, d)
        pl.when(i <= (BATCH // NBUF - 2))(lambda: fire_in(b + NBUF, d))
      return carry

    lax.fori_loop(0, BATCH // NBUF, step, 0)
    for d in range(NBUF):
      wait_out(d)

  pl.when(wid < NWORKERS - 1)(lambda: run(REG_PAIRS, REG_IN, REG_OUT))
  pl.when(wid == NWORKERS - 1)(lambda: run(TAIL_PAIRS, TAIL_IN, TAIL_OUT))


_COPY_SL = 1016
_COPY_BLK = _COPY_SL * 128


def _tc_copy_body(src_ref, dst_ref):
  dst_ref[...] = src_ref[...].reshape(8, _COPY_BLK)[:, :NOUT]


@jax.jit
def _line_filter(xf):
  mesh = plsc.VectorSubcoreMesh(core_axis_name="c", subcore_axis_name="s")
  staged = pl.kernel(
      _sc_body,
      out_type=jax.ShapeDtypeStruct((BATCH * NOUT_PAD,), jnp.float32),
      mesh=mesh,
      compiler_params=pltpu.CompilerParams(
          use_tc_tiling_on_sc=False, needs_layout_passes=False),
      scratch_types=(
          [pltpu.VMEM((IN_BUF,), jnp.float32)] * NBUF
          + [pltpu.VMEM((OUT_BUF,), jnp.float32)] * NBUF
          + [pltpu.SemaphoreType.DMA] * (2 * NBUF)
      ),
  )(xf)
  staged3 = staged.reshape(BATCH, NOUT_PAD // 128, 128)
  return pl.pallas_call(
      _tc_copy_body,
      grid=(BATCH // 8,),
      in_specs=[pl.BlockSpec((8, _COPY_SL, 128), lambda g: (g, 0, 0))],
      out_specs=pl.BlockSpec((8, NOUT), lambda g: (g, 0)),
      out_shape=jax.ShapeDtypeStruct((BATCH, NOUT), jnp.float32),
  )(staged3)


def kernel(x):
  xf = x.reshape(BATCH, IMG_H * IMG_W)
  return _line_filter(xf)
